# baseline (speedup 1.0000x reference)
import jax
import jax.numpy as jnp
from jax.experimental import pallas as pl
from jax.experimental.pallas import tpu as pltpu

_NUM_TO_REPLACE = 64
_B, _N, _C = 32, 8192, 128


def _zero_indices():
    perm = jax.random.permutation(jax.random.key(42), _N)
    return perm[:_NUM_TO_REPLACE].astype(jnp.int32)


def _body(idx_ref, pts_hbm, out_hbm, zeros_vmem, copy_sem, zero_sem):
    def batch_copy(b):
        return pltpu.make_async_copy(
            pts_hbm.at[pl.ds(b, 1)], out_hbm.at[pl.ds(b, 1)], copy_sem
        )

    for b in range(_B):
        batch_copy(b).start()
    zeros_vmem[...] = jnp.zeros((_B, 1, _C), jnp.float32)
    for b in range(_B):
        batch_copy(b).wait()

    def zero_copy(k):
        i = idx_ref[k]
        return pltpu.make_async_copy(
            zeros_vmem, out_hbm.at[:, pl.ds(i, 1), :], zero_sem
        )

    def issue(k, _):
        zero_copy(k).start()
        return 0

    jax.lax.fori_loop(0, _NUM_TO_REPLACE, issue, 0)

    def drain(k, _):
        zero_copy(k).wait()
        return 0

    jax.lax.fori_loop(0, _NUM_TO_REPLACE, drain, 0)


def kernel(pts):
    idx = _zero_indices()
    grid_spec = pltpu.PrefetchScalarGridSpec(
        num_scalar_prefetch=1,
        in_specs=[pl.BlockSpec(memory_space=pl.ANY)],
        out_specs=pl.BlockSpec(memory_space=pl.ANY),
        scratch_shapes=[
            pltpu.VMEM((_B, 1, _C), jnp.float32),
            pltpu.SemaphoreType.DMA,
            pltpu.SemaphoreType.DMA,
        ],
    )
    return pl.pallas_call(
        _body,
        grid_spec=grid_spec,
        out_shape=jax.ShapeDtypeStruct((_B, _N, _C), jnp.float32),
    )(idx, pts)


# speedup vs baseline: 30.0955x; 30.0955x over previous
import functools

import jax
import jax.numpy as jnp
from jax import lax
from jax.experimental import pallas as pl
from jax.experimental.pallas import tpu as pltpu
from jax.experimental.pallas import tpu_sc as plsc

_NUM_TO_REPLACE = 64
_B, _N, _C = 32, 8192, 128
_ROWS = _B * _N
_BLOCK_ROWS = _N


def _zero_row_ids():
    perm = jax.random.permutation(jax.random.key(42), _N)
    i_to_zero = perm[:_NUM_TO_REPLACE].astype(jnp.int32)
    rows = jnp.arange(_B, dtype=jnp.int32)[:, None] * _N + i_to_zero[None, :]
    return rows.reshape(-1)


def _copy_body(pts_ref, out_ref):
    out_ref[...] = pts_ref[...]


def _tc_copy(flat):
    return pl.pallas_call(
        _copy_body,
        grid=(_ROWS // _BLOCK_ROWS,),
        in_specs=[pl.BlockSpec((_BLOCK_ROWS, _C), lambda i: (i, 0))],
        out_specs=pl.BlockSpec((_BLOCK_ROWS, _C), lambda i: (i, 0)),
        out_shape=jax.ShapeDtypeStruct((_ROWS, _C), jnp.float32),
    )(flat)


def _sc_body(idx_hbm, zeros_hbm, out_hbm, idx_v, zeros_v, zsem):
    nc = 2
    wid = lax.axis_index("s") * nc + lax.axis_index("c")
    pltpu.sync_copy(idx_hbm.at[pl.ds(wid * _NUM_TO_REPLACE, _NUM_TO_REPLACE)],
                    idx_v)
    pltpu.sync_copy(zeros_hbm, zeros_v)
    pltpu.async_copy(zeros_v, out_hbm.at[idx_v], zsem).wait()


_sc_scatter = functools.partial(
    pl.kernel,
    out_type=(),
    mesh=plsc.VectorSubcoreMesh(core_axis_name="c", subcore_axis_name="s"),
    scratch_types=[
        pltpu.VMEM((_NUM_TO_REPLACE,), jnp.int32),
        pltpu.VMEM((_NUM_TO_REPLACE, _C), jnp.float32),
        pltpu.SemaphoreType.DMA,
    ],
)(_sc_body)


def kernel(pts):
    flat = pts.reshape(_ROWS, _C)
    idx = _zero_row_ids()
    zeros = jnp.zeros((_NUM_TO_REPLACE, _C), jnp.float32)
    out_ref = jax.new_ref(_tc_copy(flat))
    _sc_scatter(idx, zeros, out_ref)
    return out_ref[...].reshape(_B, _N, _C)


# speedup vs baseline: 30.9452x; 1.0282x over previous
import functools

import jax
import jax.numpy as jnp
from jax import lax
from jax.experimental import pallas as pl
from jax.experimental.pallas import tpu as pltpu
from jax.experimental.pallas import tpu_sc as plsc

_NUM_TO_REPLACE = 64
_B, _N, _C = 32, 8192, 128
_ROWS = _B * _N
_BLOCK_ROWS = _N


def _zero_row_ids():
    perm = jax.random.permutation(jax.random.key(42), _N)
    i_to_zero = perm[:_NUM_TO_REPLACE].astype(jnp.int32)
    rows = jnp.arange(_B, dtype=jnp.int32)[:, None] * _N + i_to_zero[None, :]
    return rows.reshape(-1)


def _copy_body(pts_ref, out_ref):
    out_ref[...] = pts_ref[...]


def _tc_copy(flat):
    return pl.pallas_call(
        _copy_body,
        grid=(_ROWS // _BLOCK_ROWS,),
        in_specs=[pl.BlockSpec((_BLOCK_ROWS, _C), lambda i: (i, 0))],
        out_specs=pl.BlockSpec((_BLOCK_ROWS, _C), lambda i: (i, 0)),
        out_shape=jax.ShapeDtypeStruct((_ROWS, _C), jnp.float32),
    )(flat)


def _sc_body(idx_hbm, out_hbm, idx_v, zeros_v, isem, zsem):
    nc = 2
    wid = lax.axis_index("s") * nc + lax.axis_index("c")
    idx_cp = pltpu.make_async_copy(
        idx_hbm.at[pl.ds(wid * _NUM_TO_REPLACE, _NUM_TO_REPLACE)], idx_v, isem)
    idx_cp.start()
    zvec = jnp.zeros((16,), jnp.float32)
    for r in range(_NUM_TO_REPLACE):
        for c in range(_C // 16):
            zeros_v[r, pl.ds(c * 16, 16)] = zvec
    idx_cp.wait()
    pltpu.async_copy(zeros_v, out_hbm.at[idx_v], zsem).wait()


_sc_scatter = functools.partial(
    pl.kernel,
    out_type=(),
    mesh=plsc.VectorSubcoreMesh(core_axis_name="c", subcore_axis_name="s"),
    scratch_types=[
        pltpu.VMEM((_NUM_TO_REPLACE,), jnp.int32),
        pltpu.VMEM((_NUM_TO_REPLACE, _C), jnp.float32),
        pltpu.SemaphoreType.DMA,
        pltpu.SemaphoreType.DMA,
    ],
)(_sc_body)


def kernel(pts):
    flat = pts.reshape(_ROWS, _C)
    idx = _zero_row_ids()
    out_ref = jax.new_ref(_tc_copy(flat))
    _sc_scatter(idx, out_ref)
    return out_ref[...].reshape(_B, _N, _C)
